# baseline (speedup 1.0000x reference)
import functools

import jax
import jax.numpy as jnp
from jax import lax
from jax.experimental import pallas as pl
from jax.experimental.pallas import tpu as pltpu
from jax.experimental.pallas import tpu_sc as plsc

N_NODES = 10000
N_EDGES = 320000
HDIM = 128

NC = 2
NS = 16
CHUNK = 112
NBUF = 3
NIDX = 6
NCHUNKS = 90
GROUP = 12
EDGES_PAD = NC * NS * NCHUNKS * CHUNK
NEPI = NCHUNKS - (NCHUNKS // GROUP) * GROUP
ACC_ROWS = 10112
ZROWS = ACC_ROWS // NS


def _sc_segment_sum(h, idx, zeros):
    mesh = plsc.VectorSubcoreMesh(core_axis_name="c", subcore_axis_name="s")

    @functools.partial(
        pl.kernel,
        out_type=jax.ShapeDtypeStruct((NC, ACC_ROWS, HDIM), jnp.float32),
        mesh=mesh,
        scratch_types=[
            pltpu.VMEM((NIDX, 2, CHUNK), jnp.int32),
            pltpu.VMEM((NBUF, CHUNK, HDIM), jnp.float32),
            pltpu.VMEM_SHARED((ACC_ROWS, HDIM), jnp.float32),
            pltpu.SemaphoreType.DMA((NIDX,)),
            pltpu.SemaphoreType.DMA((NBUF,)),
            pltpu.SemaphoreType.DMA((NBUF,)),
        ],
    )
    def k(h_hbm, idx_hbm, zero_hbm, out_hbm, idxr, rows_v, acc, isem, gsem, ssem):
        cid = lax.axis_index("c")
        sid = lax.axis_index("s")

        pltpu.sync_copy(zero_hbm.at[pl.ds(sid * ZROWS, ZROWS)],
                        acc.at[pl.ds(sid * ZROWS, ZROWS)])
        plsc.subcore_barrier()

        def fetch_idx(g, s):
            pltpu.async_copy(idx_hbm.at[:, cid, sid, g], idxr.at[s], isem.at[s])

        def wait_idx(s):
            pltpu.make_async_copy(idx_hbm.at[:, cid, sid, 0], idxr.at[s],
                                  isem.at[s]).wait()

        for c in range(5):
            fetch_idx(c, c)
        for c in range(2):
            wait_idx(c)
            pltpu.async_copy(h_hbm.at[idxr.at[c, 0]], rows_v.at[c], gsem.at[c])

        def chunk_step(g, u):
            b = u % NBUF
            bj = (u + 2) % NBUF
            sj = (u + 2) % NIDX
            su = u % NIDX
            sk = (u + 5) % NIDX
            pltpu.make_async_copy(h_hbm.at[idxr.at[su, 0]], rows_v.at[b],
                                  gsem.at[b]).wait()
            pltpu.async_copy(rows_v.at[b], acc.at[idxr.at[su, 1]],
                             ssem.at[b], add=True)

            @pl.when(jnp.logical_and(g >= 1, g + 2 < NCHUNKS))
            def _():
                pltpu.make_async_copy(rows_v.at[bj], acc.at[idxr.at[sj, 1]],
                                      ssem.at[bj]).wait()

            @pl.when(g + 2 < NCHUNKS)
            def _():
                wait_idx(sj)
                pltpu.async_copy(h_hbm.at[idxr.at[sj, 0]], rows_v.at[bj],
                                 gsem.at[bj])

            @pl.when(g + 5 < NCHUNKS)
            def _():
                fetch_idx(g + 5, sk)

        def group(G2, carry):
            for u in range(GROUP):
                chunk_step(G2 * GROUP + u, u)
            return carry

        nfull = (NCHUNKS // GROUP) * GROUP
        lax.fori_loop(0, NCHUNKS // GROUP, group, 0, unroll=False)
        for u in range(NEPI):
            chunk_step(nfull + u, (nfull + u) % GROUP)

        for g in range(NCHUNKS - NBUF, NCHUNKS):
            pltpu.make_async_copy(rows_v.at[g % NBUF],
                                  acc.at[idxr.at[g % NIDX, 1]],
                                  ssem.at[g % NBUF]).wait()

        plsc.subcore_barrier()
        pltpu.sync_copy(acc.at[pl.ds(sid * ZROWS, ZROWS)],
                        out_hbm.at[cid, pl.ds(sid * ZROWS, ZROWS)])

    return k(h, idx, zeros)


def _fused_body(x_ref, m_ref, w_ref, b_ref, p0_ref, p1_ref, u_ref, o_ref):
    xwb = jnp.dot(x_ref[...], w_ref[...], preferred_element_type=jnp.float32)
    xwb = (xwb + b_ref[...]) * m_ref[...]
    hsum = p0_ref[0] + p1_ref[0]
    h_aggr = jnp.dot(hsum, u_ref[...], preferred_element_type=jnp.float32)
    o_ref[...] = jnp.tanh(xwb + h_aggr)


def _fused_stage(x, mask2d, W_in, b2d, partials, U):
    R = 1000
    return pl.pallas_call(
        _fused_body,
        grid=(N_NODES // R,),
        in_specs=[
            pl.BlockSpec((R, HDIM), lambda i: (i, 0)),
            pl.BlockSpec((R, 1), lambda i: (i, 0)),
            pl.BlockSpec((HDIM, HDIM), lambda i: (0, 0)),
            pl.BlockSpec((1, HDIM), lambda i: (0, 0)),
            pl.BlockSpec((1, R, HDIM), lambda i: (0, i, 0)),
            pl.BlockSpec((1, R, HDIM), lambda i: (1, i, 0)),
            pl.BlockSpec((HDIM, HDIM), lambda i: (0, 0)),
        ],
        out_specs=pl.BlockSpec((R, HDIM), lambda i: (i, 0)),
        out_shape=jax.ShapeDtypeStruct((N_NODES, HDIM), jnp.float32),
    )(x, mask2d, W_in, b2d, partials, partials, U)


def kernel(x, x_mask, h, edge_index, W_in, b_in, U):
    pad = EDGES_PAD - N_EDGES
    p = jnp.arange(pad, dtype=jnp.int32)
    pad_cols = jnp.stack(
        [jnp.zeros((pad,), jnp.int32),
         N_NODES + p % (ACC_ROWS - N_NODES)])
    idx = jnp.concatenate([edge_index.astype(jnp.int32), pad_cols], axis=1)
    idx = idx.reshape(2, NC, NS, NCHUNKS, CHUNK)
    zeros = jnp.zeros((ACC_ROWS, HDIM), jnp.float32)

    partials = _sc_segment_sum(h, idx, zeros)

    mask2d = x_mask.reshape(N_NODES, 1)
    b2d = b_in.reshape(1, HDIM)
    return _fused_stage(x, mask2d, W_in, b2d, partials, U)


# speedup vs baseline: 1.6095x; 1.6095x over previous
import functools

import jax
import jax.numpy as jnp
from jax import lax
from jax.experimental import pallas as pl
from jax.experimental.pallas import tpu as pltpu
from jax.experimental.pallas import tpu_sc as plsc

N_NODES = 10000
N_EDGES = 320000
HDIM = 128

NC = 2
NS = 16
LANES = 16
CHUNK = 128
NBUF = 2
NCHUNKS = 80
EDGES_PAD = NC * NS * NCHUNKS * CHUNK
ACC_ROWS = 10112
ZROWS = ACC_ROWS // NS


def _sc_segment_sum(h, packed, zeros):
    mesh = plsc.VectorSubcoreMesh(core_axis_name="c", subcore_axis_name="s")

    @functools.partial(
        pl.kernel,
        out_type=jax.ShapeDtypeStruct((NC, ACC_ROWS, HDIM), jnp.float32),
        mesh=mesh,
        scratch_types=[
            pltpu.VMEM((NCHUNKS, CHUNK), jnp.int32),
            pltpu.VMEM((NBUF, CHUNK), jnp.int32),
            pltpu.VMEM((NBUF, CHUNK), jnp.int32),
            pltpu.VMEM((NBUF, CHUNK, HDIM), jnp.float32),
            pltpu.VMEM_SHARED((ACC_ROWS, HDIM), jnp.float32),
            pltpu.SemaphoreType.DMA((NBUF,)),
        ],
    )
    def k(h_hbm, pk_hbm, zero_hbm, out_hbm, pk_v, sidx, didx, rows_v, acc, gsem):
        cid = lax.axis_index("c")
        sid = lax.axis_index("s")

        pltpu.sync_copy(zero_hbm.at[pl.ds(sid * ZROWS, ZROWS)],
                        acc.at[pl.ds(sid * ZROWS, ZROWS)])
        pltpu.sync_copy(pk_hbm.at[cid, sid], pk_v)
        plsc.subcore_barrier()

        def unpack(j, b):
            for kk in range(CHUNK // LANES):
                pk = pk_v[j, pl.ds(kk * LANES, LANES)]
                sidx[b, pl.ds(kk * LANES, LANES)] = lax.bitwise_and(pk, 0xFFFF)
                didx[b, pl.ds(kk * LANES, LANES)] = lax.shift_right_logical(pk, 16)

        for b in range(NBUF):
            unpack(b, b)
            pltpu.async_copy(h_hbm.at[sidx.at[b]], rows_v.at[b], gsem.at[b])

        def body(g, carry):
            for b in range(NBUF):
                j = g * NBUF + b
                pltpu.make_async_copy(h_hbm.at[sidx.at[b]], rows_v.at[b],
                                      gsem.at[b]).wait()
                pltpu.sync_copy(rows_v.at[b], acc.at[didx.at[b]], add=True)

                @pl.when(g < NCHUNKS // NBUF - 1)
                def _():
                    unpack(j + NBUF, b)
                    pltpu.async_copy(h_hbm.at[sidx.at[b]], rows_v.at[b],
                                     gsem.at[b])
            return carry

        lax.fori_loop(0, NCHUNKS // NBUF, body, 0, unroll=False)

        plsc.subcore_barrier()
        pltpu.sync_copy(acc.at[pl.ds(sid * ZROWS, ZROWS)],
                        out_hbm.at[cid, pl.ds(sid * ZROWS, ZROWS)])

    return k(h, packed, zeros)


def _xwb_body(x_ref, m_ref, w_ref, b_ref, o_ref):
    h_in = jnp.dot(x_ref[...], w_ref[...], preferred_element_type=jnp.float32)
    o_ref[...] = (h_in + b_ref[...]) * m_ref[...]


def _xwb_stage(x, mask2d, W_in, b2d):
    R = 1000
    return pl.pallas_call(
        _xwb_body,
        grid=(N_NODES // R,),
        in_specs=[
            pl.BlockSpec((R, HDIM), lambda i: (i, 0)),
            pl.BlockSpec((R, 1), lambda i: (i, 0)),
            pl.BlockSpec((HDIM, HDIM), lambda i: (0, 0)),
            pl.BlockSpec((1, HDIM), lambda i: (0, 0)),
        ],
        out_specs=pl.BlockSpec((R, HDIM), lambda i: (i, 0)),
        out_shape=jax.ShapeDtypeStruct((N_NODES, HDIM), jnp.float32),
    )(x, mask2d, W_in, b2d)


def _dense_body(xwb_ref, p0_ref, p1_ref, u_ref, o_ref):
    hsum = p0_ref[...] + p1_ref[...]
    h_aggr = jnp.dot(hsum, u_ref[...], preferred_element_type=jnp.float32)
    o_ref[...] = jnp.tanh(xwb_ref[...] + h_aggr)


def _dense_stage(xwb, p0, p1, U):
    R = 1000
    grid = (N_NODES // R,)
    return pl.pallas_call(
        _dense_body,
        grid=grid,
        in_specs=[
            pl.BlockSpec((R, HDIM), lambda i: (i, 0)),
            pl.BlockSpec((R, HDIM), lambda i: (i, 0)),
            pl.BlockSpec((R, HDIM), lambda i: (i, 0)),
            pl.BlockSpec((HDIM, HDIM), lambda i: (0, 0)),
        ],
        out_specs=pl.BlockSpec((R, HDIM), lambda i: (i, 0)),
        out_shape=jax.ShapeDtypeStruct((N_NODES, HDIM), jnp.float32),
    )(xwb, p0, p1, U)


def kernel(x, x_mask, h, edge_index, W_in, b_in, U):
    src = edge_index[0].astype(jnp.int32)
    dst = edge_index[1].astype(jnp.int32)
    pad = EDGES_PAD - N_EDGES
    p = jnp.arange(pad)
    src = jnp.concatenate([src, (p % N_NODES).astype(jnp.int32)])
    dst = jnp.concatenate([dst, (N_NODES + p % (ACC_ROWS - N_NODES)
                                 ).astype(jnp.int32)])
    packed = src | (dst << 16)
    packed = packed.reshape(NC, NS, NCHUNKS, CHUNK)
    zeros = jnp.zeros((ACC_ROWS, HDIM), jnp.float32)

    partials = _sc_segment_sum(h, packed, zeros)

    mask2d = x_mask.reshape(N_NODES, 1)
    b2d = b_in.reshape(1, HDIM)
    xwb = _xwb_stage(x, mask2d, W_in, b2d)
    return _dense_stage(xwb, partials[0, :N_NODES], partials[1, :N_NODES], U)
